# baseline (speedup 1.0000x reference)
import functools

import jax
import jax.numpy as jnp
from jax import lax
from jax.experimental import pallas as pl
from jax.experimental.pallas import tpu as pltpu
from jax.experimental.pallas import tpu_sc as plsc

B, L, F, E = 4096, 200, 20, 64
N = B * L
NC, NS = 2, 16
NW = NC * NS
PER_W = N // NW
G = 128
NG = PER_W // G
R = 8

_PACK_ROWS = 4096
_PACK_GRID = N // _PACK_ROWS


def _pack_body(sal_ref, idx_ref):
    v = sal_ref[...]
    w = (jnp.int32(1) << jnp.arange(F, dtype=jnp.int32))[None, :]
    idx_ref[...] = jnp.sum(v * w, axis=1)[None, None, :]


def _pack_indices(salience_values):
    sal2d = salience_values.reshape(N, F)
    idx3d = pl.pallas_call(
        _pack_body,
        grid=(_PACK_GRID,),
        in_specs=[pl.BlockSpec((_PACK_ROWS, F), lambda i: (i, 0))],
        out_specs=pl.BlockSpec((1, 1, _PACK_ROWS), lambda i: (i, 0, 0)),
        out_shape=jax.ShapeDtypeStruct((_PACK_GRID, 1, _PACK_ROWS), jnp.int32),
    )(sal2d)
    return idx3d.reshape(NW, NG, G)


@functools.cache
def _make_gather_kernel():
    mesh = plsc.VectorSubcoreMesh(core_axis_name="c", subcore_axis_name="s")

    @functools.partial(
        pl.kernel,
        mesh=mesh,
        out_type=jax.ShapeDtypeStruct((NW * NG, G, E), jnp.float32),
        scratch_types=[
            pltpu.VMEM((NG, G), jnp.int32),
            pltpu.VMEM((R, G, E), jnp.float32),
            pltpu.SemaphoreType.DMA,
            pltpu.SemaphoreType.DMA,
        ],
        compiler_params=pltpu.CompilerParams(use_tc_tiling_on_sc=False),
    )
    def _gather_kernel(idx_hbm, table_hbm, out_hbm, idx_v, rows_v, gsem, osem):
        wid = lax.axis_index("s") * NC + lax.axis_index("c")
        pltpu.sync_copy(idx_hbm.at[wid], idx_v)

        def body(i, _):
            g0 = i * R
            gets = [
                pltpu.async_copy(table_hbm.at[idx_v.at[g0 + b]], rows_v.at[b], gsem)
                for b in range(R)
            ]
            puts = []
            for b in range(R):
                gets[b].wait()
                puts.append(
                    pltpu.async_copy(rows_v.at[b], out_hbm.at[wid * NG + g0 + b], osem)
                )
            for p in puts:
                p.wait()
            return ()

        lax.fori_loop(0, NG // R, body, (), unroll=False)

    return _gather_kernel


def kernel(salience_values, table):
    idx = _pack_indices(salience_values)
    out = _make_gather_kernel()(idx, table)
    return out.reshape(B, L, E)


# speedup vs baseline: 1.3282x; 1.3282x over previous
import functools

import jax
import jax.numpy as jnp
from jax import lax
from jax.experimental import pallas as pl
from jax.experimental.pallas import tpu as pltpu
from jax.experimental.pallas import tpu_sc as plsc

B, L, F, E = 4096, 200, 20, 64
N = B * L
NC, NS = 2, 16
NW = NC * NS
PER_W = N // NW
G = 128
NG = PER_W // G
R = 8

_PACK_BLK = 32
_PACK_GRID = (N // G) // _PACK_BLK


def _pack_body(sal_ref, idx_ref):
    v = sal_ref[...]
    w = (jnp.int32(1) << jnp.arange(F, dtype=jnp.int32))[None, None, :]
    idx_ref[...] = jnp.sum(v * w, axis=2)


def _pack_indices(salience_values):
    sal3d = salience_values.reshape(N // G, G, F)
    idx2d = pl.pallas_call(
        _pack_body,
        grid=(_PACK_GRID,),
        in_specs=[pl.BlockSpec((_PACK_BLK, G, F), lambda i: (i, 0, 0))],
        out_specs=pl.BlockSpec((_PACK_BLK, G), lambda i: (i, 0)),
        out_shape=jax.ShapeDtypeStruct((N // G, G), jnp.int32),
    )(sal3d)
    return idx2d.reshape(NW, NG, G)


@functools.cache
def _make_gather_kernel():
    mesh = plsc.VectorSubcoreMesh(core_axis_name="c", subcore_axis_name="s")

    @functools.partial(
        pl.kernel,
        mesh=mesh,
        out_type=jax.ShapeDtypeStruct((N, E), jnp.float32),
        scratch_types=[
            pltpu.VMEM((NG, G), jnp.int32),
            pltpu.VMEM((R, G, E), jnp.float32),
            pltpu.SemaphoreType.DMA,
            pltpu.SemaphoreType.DMA,
        ],
        compiler_params=pltpu.CompilerParams(use_tc_tiling_on_sc=False),
    )
    def _gather_kernel(idx_hbm, table_hbm, out_hbm, idx_v, rows_v, gsem, osem):
        wid = lax.axis_index("s") * NC + lax.axis_index("c")
        pltpu.sync_copy(idx_hbm.at[wid], idx_v)

        def body(i, _):
            g0 = i * R
            gets = [
                pltpu.async_copy(table_hbm.at[idx_v.at[g0 + b]], rows_v.at[b], gsem)
                for b in range(R)
            ]
            puts = []
            for b in range(R):
                gets[b].wait()
                puts.append(
                    pltpu.async_copy(
                        rows_v.at[b],
                        out_hbm.at[pl.ds((wid * NG + g0 + b) * G, G)],
                        osem,
                    )
                )
            for p in puts:
                p.wait()
            return ()

        lax.fori_loop(0, NG // R, body, (), unroll=False)

    return _gather_kernel


def kernel(salience_values, table):
    idx = _pack_indices(salience_values)
    out = _make_gather_kernel()(idx, table)
    return out.reshape(B, L, E)
